# baseline (speedup 1.0000x reference)
import jax
import jax.numpy as jnp
from jax import lax
from jax.experimental import pallas as pl
from jax.experimental.pallas import tpu as pltpu
from jax.experimental.pallas import tpu_sc as plsc

_B, _S, _D = 4, 4096, 1024
_info = plsc.get_sparse_core_info()
_NC = _info.num_cores
_NS = _info.num_subcores
_L = _info.num_lanes
_NW = _NC * _NS
_ROWS_PER_W = _S // _NW
_C = 32
_CHUNK = _C * _D
_NCHUNKS = _ROWS_PER_W // _C


def _sc_body(x_hbm, w_hbm, o_hbm, w_buf, x_buf):
    wid = lax.axis_index("s") * _NC + lax.axis_index("c")
    w0 = wid * (_ROWS_PER_W * _D)

    def chunk_body(ci, carry):
        wbase = pl.multiple_of(w0 + ci * _CHUNK, _CHUNK)
        pltpu.sync_copy(w_hbm.at[pl.ds(wbase, _CHUNK)], w_buf)

        def batch_body(b, carry2):
            xbase = pl.multiple_of(b * (_S * _D) + wbase, _CHUNK)
            pltpu.sync_copy(x_hbm.at[pl.ds(xbase, _CHUNK)], x_buf)

            @plsc.parallel_loop(0, _CHUNK, step=_L, unroll=8)
            def _add(i):
                x_buf[pl.ds(i, _L)] = x_buf[pl.ds(i, _L)] + w_buf[pl.ds(i, _L)]

            pltpu.sync_copy(x_buf, o_hbm.at[pl.ds(xbase, _CHUNK)])
            return carry2

        return lax.fori_loop(0, _B, batch_body, carry)

    lax.fori_loop(0, _NCHUNKS, chunk_body, 0)


def kernel(x, weight):
    xf = x.reshape(_B * _S * _D)
    wf = weight.reshape(_S * _D)
    mesh = plsc.VectorSubcoreMesh(core_axis_name="c", subcore_axis_name="s")
    out = pl.kernel(
        _sc_body,
        out_type=jax.ShapeDtypeStruct((_B * _S * _D,), jnp.float32),
        mesh=mesh,
        scratch_types=[
            pltpu.VMEM((_CHUNK,), jnp.float32),
            pltpu.VMEM((_CHUNK,), jnp.float32),
        ],
    )(xf, wf)
    return out.reshape(_B, _S, _D)


# speedup vs baseline: 2.0885x; 2.0885x over previous
import jax
import jax.numpy as jnp
from jax import lax
from jax.experimental import pallas as pl
from jax.experimental.pallas import tpu as pltpu
from jax.experimental.pallas import tpu_sc as plsc

_B, _S, _D = 4, 4096, 1024
_NC, _NS, _L = 2, 16, 16
_NW = _NC * _NS
_ROWS_PER_W = _S // _NW
_C = 32
_NCHUNKS = _ROWS_PER_W // _C


def _sc_body(x_hbm, w_hbm, o_hbm, w_buf, x_buf):
    wid = lax.axis_index("s") * _NC + lax.axis_index("c")
    row0 = wid * _ROWS_PER_W

    def chunk_body(ci, carry):
        rows = pl.multiple_of(row0 + ci * _C, _C)
        pltpu.sync_copy(w_hbm.at[pl.ds(rows, _C), :], w_buf)

        def batch_body(b, carry2):
            pltpu.sync_copy(x_hbm.at[b, pl.ds(rows, _C), :], x_buf)

            def row_body(r, carry3):
                @plsc.parallel_loop(0, _D, step=_L, unroll=8)
                def _add(i):
                    x_buf[r, pl.ds(i, _L)] = (
                        x_buf[r, pl.ds(i, _L)] + w_buf[r, pl.ds(i, _L)]
                    )

                return carry3

            lax.fori_loop(0, _C, row_body, 0)
            pltpu.sync_copy(x_buf, o_hbm.at[b, pl.ds(rows, _C), :])
            return carry2

        return lax.fori_loop(0, _B, batch_body, carry)

    lax.fori_loop(0, _NCHUNKS, chunk_body, 0)


def kernel(x, weight):
    mesh = plsc.VectorSubcoreMesh(core_axis_name="c", subcore_axis_name="s")
    out = pl.kernel(
        _sc_body,
        out_type=jax.ShapeDtypeStruct((_B, _S, _D), jnp.float32),
        mesh=mesh,
        scratch_types=[
            pltpu.VMEM((_C, _D), jnp.float32),
            pltpu.VMEM((_C, _D), jnp.float32),
        ],
    )(x, weight)
    return out


# speedup vs baseline: 3.1559x; 1.5111x over previous
import jax
import jax.numpy as jnp
from jax import lax
from jax.experimental import pallas as pl
from jax.experimental.pallas import tpu as pltpu
from jax.experimental.pallas import tpu_sc as plsc

_B, _S, _D = 4, 4096, 1024
_NC, _NS, _L = 2, 16, 16
_NW = _NC * _NS
_ROWS_PER_W = _S // _NW
_C = 16
_NCH = _ROWS_PER_W // _C
_JOBS = [(ci, b) for ci in range(_NCH) for b in range(_B)]


def _sc_body(x_hbm, w_hbm, o_hbm,
             wb0, wb1, xb0, xb1, xb2,
             sw0, sw1, si0, si1, si2, so0, so1, so2):
    wbufs = (wb0, wb1)
    xbufs = (xb0, xb1, xb2)
    sws = (sw0, sw1)
    sis = (si0, si1, si2)
    sos = (so0, so1, so2)

    wid = lax.axis_index("s") * _NC + lax.axis_index("c")
    row0 = wid * _ROWS_PER_W

    def rows(ci):
        return pl.multiple_of(row0 + ci * _C, _C)

    def start_w(ci):
        pltpu.async_copy(w_hbm.at[pl.ds(rows(ci), _C), :],
                         wbufs[ci % 2], sws[ci % 2])

    def wait_w(ci):
        pltpu.make_async_copy(w_hbm.at[pl.ds(rows(ci), _C), :],
                              wbufs[ci % 2], sws[ci % 2]).wait()

    def start_in(j):
        ci, b = _JOBS[j]
        pltpu.async_copy(x_hbm.at[b, pl.ds(rows(ci), _C), :],
                         xbufs[j % 3], sis[j % 3])

    def wait_in(j):
        ci, b = _JOBS[j]
        pltpu.make_async_copy(x_hbm.at[b, pl.ds(rows(ci), _C), :],
                              xbufs[j % 3], sis[j % 3]).wait()

    def start_out(j):
        ci, b = _JOBS[j]
        pltpu.async_copy(xbufs[j % 3],
                         o_hbm.at[b, pl.ds(rows(ci), _C), :], sos[j % 3])

    def wait_out(j):
        ci, b = _JOBS[j]
        pltpu.make_async_copy(xbufs[j % 3],
                              o_hbm.at[b, pl.ds(rows(ci), _C), :],
                              sos[j % 3]).wait()

    start_w(0)
    start_in(0)
    start_in(1)

    for j, (ci, b) in enumerate(_JOBS):
        if b == 0 and ci + 1 < _NCH:
            start_w(ci + 1)
        jn = j + 2
        if jn < len(_JOBS):
            if jn - 3 >= 0:
                wait_out(jn - 3)
            start_in(jn)
        if b == 0:
            wait_w(ci)
        wait_in(j)

        xb = xbufs[j % 3]
        wb = wbufs[ci % 2]

        def row_body(r, carry):
            @plsc.parallel_loop(0, _D, step=_L, unroll=8)
            def _add(i):
                xb[r, pl.ds(i, _L)] = xb[r, pl.ds(i, _L)] + wb[r, pl.ds(i, _L)]

            return carry

        lax.fori_loop(0, _C, row_body, 0)
        start_out(j)

    for j in range(len(_JOBS) - 3, len(_JOBS)):
        wait_out(j)


def kernel(x, weight):
    mesh = plsc.VectorSubcoreMesh(core_axis_name="c", subcore_axis_name="s")
    out = pl.kernel(
        _sc_body,
        out_type=jax.ShapeDtypeStruct((_B, _S, _D), jnp.float32),
        mesh=mesh,
        scratch_types=[
            pltpu.VMEM((_C, _D), jnp.float32),
            pltpu.VMEM((_C, _D), jnp.float32),
            pltpu.VMEM((_C, _D), jnp.float32),
            pltpu.VMEM((_C, _D), jnp.float32),
            pltpu.VMEM((_C, _D), jnp.float32),
            pltpu.SemaphoreType.DMA,
            pltpu.SemaphoreType.DMA,
            pltpu.SemaphoreType.DMA,
            pltpu.SemaphoreType.DMA,
            pltpu.SemaphoreType.DMA,
            pltpu.SemaphoreType.DMA,
            pltpu.SemaphoreType.DMA,
            pltpu.SemaphoreType.DMA,
        ],
    )(x, weight)
    return out
